# Initial kernel scaffold; baseline (speedup 1.0000x reference)
import jax
import jax.numpy as jnp
from jax.experimental import pallas as pl


def kernel(x, conv_w_0, conv_b_0, conv_w_1, conv_b_1, conv_w_2, conv_b_2, conv_w_3, conv_b_3, conv_w_4, conv_b_4, conv_w_5, conv_b_5, conv_w_6, conv_b_6, conv_w_7, conv_b_7, conv_w_8, conv_b_8, conv_w_9, conv_b_9, outa_w_0, outa_b_0, outa_w_1, outa_b_1, outa_w_2, outa_b_2, outc_w_0, outc_b_0, outc_w_1, outc_b_1, outc_w_2, outc_b_2):
    raise NotImplementedError("write your pallas kernel here")


# speedup vs baseline: 2.2293x; 2.2293x over previous
import functools

import jax
import jax.numpy as jnp
from jax.experimental import pallas as pl
from jax.experimental.pallas import tpu as pltpu

_VMEM_LIMIT = 48 * 1024 * 1024


def _dot(a, b):
    return jnp.dot(a, b, preferred_element_type=jnp.float32)


def _block_kernel(x_ref, w1_ref, b1_ref, w2_ref, b2_ref, o_ref,
                  *, h, w, cin, c1, c2, mode1, mode2):
    wp = w + 2
    m = h * wp
    x = x_ref[0]

    if mode1 == "full":
        xc = jnp.concatenate(
            [x[dy * wp + dx: dy * wp + dx + m, :] for dy in range(3) for dx in range(3)],
            axis=1)
        acc1 = _dot(xc, w1_ref[...])
    elif mode1 == "xc":
        xc = jnp.concatenate(
            [x[0: m + 2], x[wp: wp + m + 2], x[2 * wp: 2 * wp + m + 2]], axis=1)
        acc1 = _dot(xc[0:m], w1_ref[0])
        acc1 = acc1 + _dot(xc[1:m + 1], w1_ref[1])
        acc1 = acc1 + _dot(xc[2:m + 2], w1_ref[2])
    else:
        acc1 = _dot(x[0:m], w1_ref[0])
        for t in range(1, 9):
            dy, dx = divmod(t, 3)
            off = dy * wp + dx
            acc1 = acc1 + _dot(x[off:off + m], w1_ref[t])

    y1 = jnp.maximum(acc1 + b1_ref[...], 0.0)
    col = jax.lax.broadcasted_iota(jnp.int32, (m, c1), 0) % wp
    y1 = jnp.where(col < w, y1, 0.0).astype(jnp.bfloat16)

    z = jnp.zeros((wp + 1, c1), jnp.bfloat16)
    if mode2 == "xc":
        blk0 = jnp.concatenate([z, y1[0: m + 1 - wp]], axis=0)
        blk1 = jnp.concatenate([z[:1], y1, z[:1]], axis=0)
        blk2 = jnp.concatenate([y1[wp - 1: m], z], axis=0)
        xc2 = jnp.concatenate([blk0, blk1, blk2], axis=1)
        acc2 = _dot(xc2[0:m], w2_ref[0])
        acc2 = acc2 + _dot(xc2[1:m + 1], w2_ref[1])
        acc2 = acc2 + _dot(xc2[2:m + 2], w2_ref[2])
    else:
        ypad = jnp.concatenate([z, y1, z], axis=0)
        acc2 = _dot(ypad[0:m], w2_ref[0])
        for t in range(1, 9):
            dy, dx = divmod(t, 3)
            off = dy * wp + dx
            acc2 = acc2 + _dot(ypad[off:off + m], w2_ref[t])

    y2 = jnp.maximum(acc2 + b2_ref[...], 0.0).astype(jnp.bfloat16)

    r = y2.reshape(h // 2, 2 * wp, c2)
    va = jnp.maximum(r[:, :wp, :], r[:, wp:, :])
    q = va.reshape((h // 2) * (wp // 2), 2, c2)
    pooled = jnp.maximum(q[:, 0, :], q[:, 1, :])

    wq = wp // 2
    colp = jax.lax.broadcasted_iota(jnp.int32, ((h // 2) * wq, c2), 0) % wq
    o_ref[0] = jnp.where(colp < w // 2, pooled, jnp.bfloat16(0.0))


def _run_block(xs, w1, b1, w2, b2, *, h, w):
    n = xs.shape[0]
    cin = xs.shape[-1]
    c1 = w1.shape[-1]
    c2 = w2.shape[-1]
    wp = w + 2

    def prep(w9, mode):
        cc = w9.shape[1]
        co = w9.shape[2]
        if mode == "full":
            return w9.reshape(9 * cc, co)
        if mode == "xc":
            return w9.reshape(3, 3, cc, co).transpose(1, 0, 2, 3).reshape(3, 3 * cc, co)
        return w9

    mode1 = "full" if cin <= 16 else ("xc" if cin < 256 else "taps")
    mode2 = "xc" if c1 < 256 else "taps"
    w1p = prep(w1, mode1)
    w2p = prep(w2, mode2)

    out = pl.pallas_call(
        functools.partial(_block_kernel, h=h, w=w, cin=cin, c1=c1, c2=c2,
                          mode1=mode1, mode2=mode2),
        out_shape=jax.ShapeDtypeStruct((n, (h // 2) * (wp // 2), c2), jnp.bfloat16),
        grid=(n,),
        in_specs=[
            pl.BlockSpec((1, (h + 3) * wp, cin), lambda i: (i, 0, 0)),
            pl.BlockSpec(w1p.shape, lambda i, _nd=w1p.ndim: (0,) * _nd),
            pl.BlockSpec((1, c1), lambda i: (0, 0)),
            pl.BlockSpec(w2p.shape, lambda i, _nd=w2p.ndim: (0,) * _nd),
            pl.BlockSpec((1, c2), lambda i: (0, 0)),
        ],
        out_specs=pl.BlockSpec((1, (h // 2) * (wp // 2), c2), lambda i: (i, 0, 0)),
        compiler_params=pltpu.CompilerParams(
            dimension_semantics=("parallel",),
            vmem_limit_bytes=_VMEM_LIMIT,
        ),
    )(xs, w1p, b1, w2p, b2)
    return out


def _restrip(pooled, h2, w2c):
    n, _, c = pooled.shape
    t = pooled.reshape(n, h2, w2c + 1, c)
    tp = jnp.pad(t, ((0, 0), (1, 2), (1, 0), (0, 0)))
    return tp.reshape(n, (h2 + 3) * (w2c + 2), c)


def _mm_kernel(x_ref, w_ref, b_ref, o_ref):
    y = _dot(x_ref[...], w_ref[...]) + b_ref[...]
    o_ref[...] = jnp.maximum(y, 0.0).astype(o_ref.dtype)


def _im2col_s2(xp, ho, wo):
    n = xp.shape[0]
    c = xp.shape[-1]
    taps = []
    for dy in range(3):
        for dx in range(3):
            taps.append(jax.lax.slice(
                xp, (0, dy, dx, 0),
                (n, dy + 2 * (ho - 1) + 1, dx + 2 * (wo - 1) + 1, c),
                (1, 2, 2, 1)))
    return jnp.concatenate(taps, axis=-1).reshape(n * ho * wo, 9 * c)


def _conv_s2(patches, w9, b):
    mrows, k = patches.shape
    co = w9.shape[-1]
    return pl.pallas_call(
        _mm_kernel,
        out_shape=jax.ShapeDtypeStruct((mrows, co), jnp.bfloat16),
        grid=(2,),
        in_specs=[
            pl.BlockSpec((mrows // 2, k), lambda i: (i, 0)),
            pl.BlockSpec((k, co), lambda i: (0, 0)),
            pl.BlockSpec((1, co), lambda i: (0, 0)),
        ],
        out_specs=pl.BlockSpec((mrows // 2, co), lambda i: (i, 0)),
        compiler_params=pltpu.CompilerParams(
            dimension_semantics=("parallel",),
            vmem_limit_bytes=_VMEM_LIMIT,
        ),
    )(patches, w9.reshape(k, co), b)


def _tail_kernel(p_ref, w_ref, b_ref,
                 wa1, ba1, wa2, ba2, wa3, ba3,
                 wc1, bc1, wc2, bc2, wc3, bc3,
                 a_ref, c_ref, *, n):
    y = _dot(p_ref[...], w_ref[...]) + b_ref[...]
    y = jnp.maximum(y, 0.0).astype(jnp.bfloat16)
    feat = jnp.max(y.reshape(n, 4, y.shape[-1]), axis=1).astype(jnp.float32)

    h1 = jnp.maximum(_dot(feat, wa1[...]) + ba1[...], 0.0)
    h1 = jnp.maximum(_dot(h1, wa2[...]) + ba2[...], 0.0)
    a = _dot(h1, wa3[...]) + ba3[...]
    a = a - jnp.max(a, axis=-1, keepdims=True)
    e = jnp.exp(a)
    a_ref[...] = e / jnp.sum(e, axis=-1, keepdims=True)

    g = jnp.maximum(_dot(feat, wc1[...]) + bc1[...], 0.0)
    g = jnp.maximum(_dot(g, wc2[...]) + bc2[...], 0.0)
    c_ref[...] = _dot(g, wc3[...]) + bc3[...]


def _tail(patches, w9, b, pa, pc):
    n = patches.shape[0] // 4
    k = patches.shape[1]
    co = w9.shape[-1]
    act_num = pa[-1][0].shape[1]

    args = [patches, w9.reshape(k, co), b]
    for wgt, bia in list(pa) + list(pc):
        args += [wgt, bia]

    def full_spec(arr):
        nd = arr.ndim
        return pl.BlockSpec(arr.shape, lambda i, _nd=nd: (0,) * _nd)

    a, c = pl.pallas_call(
        functools.partial(_tail_kernel, n=n),
        out_shape=(jax.ShapeDtypeStruct((n, act_num), jnp.float32),
                   jax.ShapeDtypeStruct((n, 1), jnp.float32)),
        grid=(1,),
        in_specs=[full_spec(arr) for arr in args],
        out_specs=(pl.BlockSpec((n, act_num), lambda i: (0, 0)),
                   pl.BlockSpec((n, 1), lambda i: (0, 0))),
        compiler_params=pltpu.CompilerParams(
            dimension_semantics=("arbitrary",),
            vmem_limit_bytes=_VMEM_LIMIT,
        ),
    )(*args)
    return a, c


def kernel(x,
           conv_w_0, conv_b_0, conv_w_1, conv_b_1, conv_w_2, conv_b_2,
           conv_w_3, conv_b_3, conv_w_4, conv_b_4, conv_w_5, conv_b_5,
           conv_w_6, conv_b_6, conv_w_7, conv_b_7, conv_w_8, conv_b_8,
           conv_w_9, conv_b_9,
           outa_w_0, outa_b_0, outa_w_1, outa_b_1, outa_w_2, outa_b_2,
           outc_w_0, outc_b_0, outc_w_1, outc_b_1, outc_w_2, outc_b_2):
    n = x.shape[0]
    sp = x.shape[-1]

    xh = jnp.transpose(x, (0, 2, 3, 1)).astype(jnp.bfloat16)
    cin_pad = conv_w_0.shape[1]
    xh = jnp.pad(xh, ((0, 0), (0, 0), (0, 0), (0, cin_pad - xh.shape[-1])))
    xp = jnp.pad(xh, ((0, 0), (1, 2), (1, 1), (0, 0)))
    xs = xp.reshape(n, (sp + 3) * (sp + 2), cin_pad)

    h = sp
    cw = [(conv_w_0, conv_b_0), (conv_w_1, conv_b_1), (conv_w_2, conv_b_2),
          (conv_w_3, conv_b_3), (conv_w_4, conv_b_4), (conv_w_5, conv_b_5),
          (conv_w_6, conv_b_6), (conv_w_7, conv_b_7)]
    for blk in range(4):
        (w1, b1), (w2, b2) = cw[2 * blk], cw[2 * blk + 1]
        out = _run_block(xs, w1, b1, w2, b2, h=h, w=h)
        h //= 2
        if blk < 3:
            xs = _restrip(out, h, h)

    c4 = conv_w_7.shape[-1]
    t = out.reshape(n, h, h + 1, c4)
    tp = jnp.pad(t, ((0, 0), (1, 1), (1, 0), (0, 0)))

    ho = h // 2
    p1 = _im2col_s2(tp, ho, ho)
    y5 = _conv_s2(p1, conv_w_8, conv_b_8)

    y5 = y5.reshape(n, ho, ho, c4)
    y5p = jnp.pad(y5, ((0, 0), (1, 1), (1, 1), (0, 0)))
    p2 = _im2col_s2(y5p, ho // 2, ho // 2)

    return _tail(p2, conv_w_9, conv_b_9,
                 [(outa_w_0, outa_b_0), (outa_w_1, outa_b_1), (outa_w_2, outa_b_2)],
                 [(outc_w_0, outc_b_0), (outc_w_1, outc_b_1), (outc_w_2, outc_b_2)])
